# Initial kernel scaffold; baseline (speedup 1.0000x reference)
import functools

import jax
import jax.numpy as jnp
from jax import lax
from jax.experimental import pallas as pl
from jax.experimental.pallas import tpu as pltpu
from jax.experimental.pallas import tpu_sc as plsc

VOCAB = 100000
EMB = 128
HID = 512
B = 4096
L = 200
EPS = 1e-5

CHUNK = 100
VREGS = EMB // 16


@functools.lru_cache(maxsize=None)
def _sc_pool():
    info = plsc.get_sparse_core_info()
    nc, ns = info.num_cores, info.num_subcores
    nw = nc * ns
    spw = B // nw

    mesh = plsc.VectorSubcoreMesh(core_axis_name="c", subcore_axis_name="s")

    @functools.partial(
        pl.kernel,
        mesh=mesh,
        out_type=jax.ShapeDtypeStruct((B, EMB), jnp.float32),
        scratch_types=[
            pltpu.VMEM((spw, L), jnp.int32),
            pltpu.VMEM((2, CHUNK, EMB), jnp.float32),
            pltpu.VMEM((spw, EMB), jnp.float32),
            pltpu.SemaphoreType.DMA,
        ],
    )
    def pool(x_hbm, emb_hbm, out_hbm, idx_v, rows_v, out_v, sem):
        c = lax.axis_index("c")
        s = lax.axis_index("s")
        wid = s * nc + c
        base = wid * spw
        pltpu.sync_copy(x_hbm.at[pl.ds(base, spw)], idx_v)

        def sample(i, carry):
            cp0 = pltpu.async_copy(
                emb_hbm.at[idx_v.at[i, pl.ds(0, CHUNK)]], rows_v.at[0], sem)
            cp1 = pltpu.async_copy(
                emb_hbm.at[idx_v.at[i, pl.ds(CHUNK, CHUNK)]], rows_v.at[1], sem)
            cp0.wait()
            cp1.wait()

            def make_body(buf):
                def body(j, acc):
                    return tuple(
                        acc[k] + rows_v[buf, j, pl.ds(16 * k, 16)]
                        for k in range(VREGS))
                return body

            acc = tuple(jnp.zeros((16,), jnp.float32) for _ in range(VREGS))
            acc = lax.fori_loop(0, CHUNK, make_body(0), acc)
            acc = lax.fori_loop(0, CHUNK, make_body(1), acc)
            for k in range(VREGS):
                out_v[i, pl.ds(16 * k, 16)] = acc[k]
            return carry

        lax.fori_loop(0, spw, sample, 0)
        pltpu.sync_copy(out_v, out_hbm.at[pl.ds(base, spw)])

    return pool


def _dense_body(e_ref, w_ref, b_ref, g_ref, bt_ref, out_ref):
    e = e_ref[...]
    w = w_ref[...]
    z = lax.dot_general(e, w, (((1,), (1,)), ((), ())),
                        preferred_element_type=jnp.float32)
    h = jax.nn.sigmoid(z * (1.0 / L) + b_ref[...])
    mu = jnp.mean(h, axis=0, keepdims=True)
    var = jnp.mean((h - mu) ** 2, axis=0, keepdims=True)
    out_ref[...] = (h - mu) * lax.rsqrt(var + EPS) * g_ref[...] + bt_ref[...]


def _tc_dense(e_sum, w_h, b_h, gamma, beta):
    return pl.pallas_call(
        _dense_body,
        out_shape=jax.ShapeDtypeStruct((B, HID), jnp.float32),
    )(e_sum, w_h, b_h.reshape(1, HID), gamma.reshape(1, HID),
      beta.reshape(1, HID))


def kernel(x, emb, W_h, b_h, gamma, beta):
    x = x.astype(jnp.int32)
    e_sum = _sc_pool()(x, emb)
    return _tc_dense(e_sum, W_h, b_h, gamma, beta)


# speedup vs baseline: 12.8929x; 12.8929x over previous
import functools

import jax
import jax.numpy as jnp
from jax import lax
from jax.experimental import pallas as pl
from jax.experimental.pallas import tpu as pltpu
from jax.experimental.pallas import tpu_sc as plsc

VOCAB = 100000
EMB = 128
HID = 512
B = 4096
L = 200
EPS = 1e-5

CHUNK0 = 120
CHUNK1 = L - CHUNK0
VREGS = EMB // 16


@functools.lru_cache(maxsize=None)
def _sc_pool():
    info = plsc.get_sparse_core_info()
    nc, ns = info.num_cores, info.num_subcores
    nw = nc * ns
    spw = B // nw

    mesh = plsc.VectorSubcoreMesh(core_axis_name="c", subcore_axis_name="s")

    @functools.partial(
        pl.kernel,
        mesh=mesh,
        out_type=jax.ShapeDtypeStruct((B * EMB,), jnp.float32),
        scratch_types=[
            pltpu.VMEM((spw * L,), jnp.int32),
            pltpu.VMEM((2 * L, EMB), jnp.float32),
            pltpu.VMEM((spw * EMB,), jnp.float32),
            pltpu.SemaphoreType.DMA,
            pltpu.SemaphoreType.DMA,
        ],
    )
    def pool(x_hbm, emb_hbm, out_hbm, idx_v, rows_v, out_v, sem0, sem1):
        c = lax.axis_index("c")
        s = lax.axis_index("s")
        wid = s * nc + c
        pltpu.sync_copy(x_hbm.at[pl.ds(wid * (spw * L), spw * L)], idx_v)

        def issue(i, slot, sem):
            pltpu.async_copy(
                emb_hbm.at[idx_v.at[pl.ds(i * L, CHUNK0)]],
                rows_v.at[pl.ds(slot * L, CHUNK0)], sem)
            pltpu.async_copy(
                emb_hbm.at[idx_v.at[pl.ds(i * L + CHUNK0, CHUNK1)]],
                rows_v.at[pl.ds(slot * L + CHUNK0, CHUNK1)], sem)

        def drain(i, slot, sem):
            pltpu.make_async_copy(
                emb_hbm.at[idx_v.at[pl.ds(i * L, CHUNK0)]],
                rows_v.at[pl.ds(slot * L, CHUNK0)], sem).wait()
            pltpu.make_async_copy(
                emb_hbm.at[idx_v.at[pl.ds(i * L + CHUNK0, CHUNK1)]],
                rows_v.at[pl.ds(slot * L + CHUNK0, CHUNK1)], sem).wait()

        def reduce_store(i, slot):
            def body(j, acc):
                return tuple(
                    acc[k] + rows_v[slot * L + j, pl.ds(16 * k, 16)]
                    for k in range(VREGS))
            acc = tuple(jnp.zeros((16,), jnp.float32) for _ in range(VREGS))
            acc = lax.fori_loop(0, L, body, acc)
            for k in range(VREGS):
                out_v[pl.ds(i * EMB + 16 * k, 16)] = acc[k]

        issue(0, 0, sem0)

        def pair(p, carry):
            i0 = 2 * p
            issue(i0 + 1, 1, sem1)
            drain(i0, 0, sem0)
            reduce_store(i0, 0)

            @pl.when(i0 + 2 < spw)
            def _():
                issue(i0 + 2, 0, sem0)

            drain(i0 + 1, 1, sem1)
            reduce_store(i0 + 1, 1)
            return carry

        lax.fori_loop(0, spw // 2, pair, 0)
        pltpu.sync_copy(out_v, out_hbm.at[pl.ds(wid * (spw * EMB), spw * EMB)])

    return pool


def _dense_body(e_ref, w_ref, b_ref, g_ref, bt_ref, out_ref):
    e = e_ref[...]
    w = w_ref[...]
    z = lax.dot_general(e, w, (((1,), (1,)), ((), ())),
                        preferred_element_type=jnp.float32)
    h = jax.nn.sigmoid(z * (1.0 / L) + b_ref[...])
    mu = jnp.mean(h, axis=0, keepdims=True)
    var = jnp.mean((h - mu) ** 2, axis=0, keepdims=True)
    out_ref[...] = (h - mu) * lax.rsqrt(var + EPS) * g_ref[...] + bt_ref[...]


def _tc_dense(e_sum, w_h, b_h, gamma, beta):
    return pl.pallas_call(
        _dense_body,
        out_shape=jax.ShapeDtypeStruct((B, HID), jnp.float32),
    )(e_sum, w_h, b_h.reshape(1, HID), gamma.reshape(1, HID),
      beta.reshape(1, HID))


def kernel(x, emb, W_h, b_h, gamma, beta):
    x = x.astype(jnp.int32).reshape(B * L)
    e_sum = _sc_pool()(x, emb).reshape(B, EMB)
    return _tc_dense(e_sum, W_h, b_h, gamma, beta)
